# initial kernel scaffold (unmeasured)
import jax
import jax.numpy as jnp
from jax import lax
from jax.experimental import pallas as pl
from jax.experimental.pallas import tpu as pltpu

N_DEV = 16


def kernel(x, dy, gamma):
    m, d = x.shape

    def body(x_ref, dy_ref, gamma_ref, out_ref, partial_ref, gather_ref,
             send_sems, recv_sems):
        my = lax.axis_index("i")

        xv = x_ref[:, :]
        dyv = dy_ref[:, :]
        mu = jnp.mean(xv, axis=1, keepdims=True)
        xc = xv - mu
        var = jnp.mean(xc * xc, axis=1, keepdims=True)
        rstd = lax.rsqrt(var + 1e-5)
        xhat = xc * rstd
        dgamma = jnp.sum(dyv * xhat, axis=0)
        dbeta = jnp.sum(dyv, axis=0)
        partial_ref[:, :] = jnp.stack([dgamma, dbeta])

        gather_ref[pl.ds(my, 1), :, :] = partial_ref[:, :][None, :, :]

        for s in range(N_DEV):
            @pl.when(my != s)
            def _(s=s):
                rdma = pltpu.make_async_remote_copy(
                    src_ref=partial_ref,
                    dst_ref=gather_ref.at[my],
                    send_sem=send_sems.at[s],
                    recv_sem=recv_sems.at[my],
                    device_id=(s,),
                    device_id_type=pl.DeviceIdType.MESH,
                )
                rdma.start()

        for s in range(N_DEV):
            @pl.when(my != s)
            def _(s=s):
                rdma = pltpu.make_async_remote_copy(
                    src_ref=partial_ref,
                    dst_ref=gather_ref.at[s],
                    send_sem=send_sems.at[s],
                    recv_sem=recv_sems.at[s],
                    device_id=(s,),
                    device_id_type=pl.DeviceIdType.MESH,
                )
                rdma.wait_recv()
                rdma.wait_send()

        out_ref[:, :] = jnp.sum(gather_ref[:, :, :], axis=0)

    return pl.pallas_call(
        body,
        out_shape=jax.ShapeDtypeStruct((2, d), jnp.float32),
        in_specs=[
            pl.BlockSpec(memory_space=pltpu.VMEM),
            pl.BlockSpec(memory_space=pltpu.VMEM),
            pl.BlockSpec(memory_space=pltpu.ANY),
        ],
        out_specs=pl.BlockSpec(memory_space=pltpu.VMEM),
        scratch_shapes=[
            pltpu.VMEM((2, d), jnp.float32),
            pltpu.VMEM((N_DEV, 2, d), jnp.float32),
            pltpu.SemaphoreType.DMA((N_DEV,)),
            pltpu.SemaphoreType.DMA((N_DEV,)),
        ],
        compiler_params=pltpu.CompilerParams(collective_id=0),
    )(x, dy, gamma)


# baseline (device time: 32234 ns/iter reference)
import jax
import jax.numpy as jnp
from jax import lax
from jax.experimental import pallas as pl
from jax.experimental.pallas import tpu as pltpu

N_DEV = 16


def kernel(x, dy, gamma):
    m, d = x.shape

    def body(x_ref, dy_ref, gamma_ref, out_ref, partial_ref, gather_ref,
             send_sems, recv_sems):
        my = lax.axis_index("i")

        xv = x_ref[:, :]
        dyv = dy_ref[:, :]
        mu = jnp.mean(xv, axis=1, keepdims=True)
        xc = xv - mu
        var = jnp.mean(xc * xc, axis=1, keepdims=True)
        rstd = lax.rsqrt(var + 1e-5)
        xhat = xc * rstd
        dgamma = jnp.sum(dyv * xhat, axis=0)
        dbeta = jnp.sum(dyv, axis=0)
        partial_ref[:, :] = jnp.stack([dgamma, dbeta])

        gather_ref[pl.ds(my, 1), :, :] = partial_ref[:, :][None, :, :]

        for s in range(N_DEV):
            @pl.when(my != s)
            def _(s=s):
                rdma = pltpu.make_async_remote_copy(
                    src_ref=partial_ref,
                    dst_ref=gather_ref.at[my],
                    send_sem=send_sems.at[s],
                    recv_sem=recv_sems.at[my],
                    device_id=(s,),
                    device_id_type=pl.DeviceIdType.MESH,
                )
                rdma.start()

        for s in range(N_DEV):
            @pl.when(my != s)
            def _(s=s):
                rdma = pltpu.make_async_remote_copy(
                    src_ref=partial_ref,
                    dst_ref=gather_ref.at[s],
                    send_sem=send_sems.at[s],
                    recv_sem=recv_sems.at[s],
                    device_id=(s,),
                    device_id_type=pl.DeviceIdType.MESH,
                )
                rdma.wait_recv()
                rdma.wait_send()

        out_ref[:, :] = jnp.sum(gather_ref[:, :, :], axis=0)

    return pl.pallas_call(
        body,
        out_shape=jax.ShapeDtypeStruct((2, d), jnp.float32),
        in_specs=[
            pl.BlockSpec(memory_space=pltpu.VMEM),
            pl.BlockSpec(memory_space=pltpu.VMEM),
            pl.BlockSpec(memory_space=pl.ANY),
        ],
        out_specs=pl.BlockSpec(memory_space=pltpu.VMEM),
        scratch_shapes=[
            pltpu.VMEM((2, d), jnp.float32),
            pltpu.VMEM((N_DEV, 2, d), jnp.float32),
            pltpu.SemaphoreType.DMA((N_DEV,)),
            pltpu.SemaphoreType.DMA((N_DEV,)),
        ],
    )(x, dy, gamma)
